# Initial kernel scaffold; baseline (speedup 1.0000x reference)
import jax
import jax.numpy as jnp
from jax.experimental import pallas as pl


def kernel(listing_id, listing_industry_type, employer_num_employees, listing_loc_latitude, listing_loc_longitude, listing_skills, listing_table, industry_table):
    raise NotImplementedError("write your pallas kernel here")


# speedup vs baseline: 2.2495x; 2.2495x over previous
import functools

import jax
import jax.numpy as jnp
from jax import lax
from jax.experimental import pallas as pl
from jax.experimental.pallas import tpu as pltpu
from jax.experimental.pallas import tpu_sc as plsc

B = 16384
EMB = 32
SK = 20
OUT_W = 568
NC = 2
NS = 16
L = 16
NW = NC * NS
ROWS_PER_W = B // NW
R = 64
NCHUNK = ROWS_PER_W // R


def _sc_body(lid_hbm, ind_hbm, emp_hbm, lat_hbm, lon_hbm, sk_hbm,
             ltab_hbm, itab_hbm, out_hbm,
             lid_v, ind_v, emp_v, lat_v, lon_v, sk_v,
             lrows_v, irows_v, out_v, sem):
    wid = lax.axis_index("s") * NC + lax.axis_index("c")
    iota = lax.iota(jnp.int32, L)
    ones = jnp.full((L,), 1.0, jnp.float32)
    zeros = jnp.zeros((L,), jnp.float32)

    col_starts = [64 + 16 * t for t in range(31)] + [OUT_W - 16]

    def zero_row(r, _):
        for c in col_starts:
            out_v[r, pl.ds(c, L)] = zeros
        return 0

    lax.fori_loop(0, R, zero_row, 0)

    def do_chunk(i, _):
        base = wid * ROWS_PER_W + i * R
        pltpu.sync_copy(lid_hbm.at[pl.ds(base, R)], lid_v)
        pltpu.sync_copy(ind_hbm.at[pl.ds(base, R)], ind_v)
        pltpu.sync_copy(emp_hbm.at[pl.ds(base, R)], emp_v)
        pltpu.sync_copy(lat_hbm.at[pl.ds(base, R)], lat_v)
        pltpu.sync_copy(lon_hbm.at[pl.ds(base, R)], lon_v)
        pltpu.sync_copy(sk_hbm.at[pl.ds(base, R)], sk_v)
        pltpu.async_copy(ltab_hbm.at[lid_v], lrows_v, sem).wait()
        pltpu.async_copy(itab_hbm.at[ind_v], irows_v, sem).wait()

        for g in range(R // L):
            r0 = g * L
            rows = r0 + iota
            for j in range(L):
                r = r0 + j
                out_v[r, pl.ds(0, L)] = lrows_v[r, pl.ds(0, L)]
                out_v[r, pl.ds(L, L)] = lrows_v[r, pl.ds(L, L)]
                out_v[r, pl.ds(32, L)] = irows_v[r, pl.ds(0, L)]
                out_v[r, pl.ds(48, L)] = irows_v[r, pl.ds(L, L)]
            plsc.store_scatter(out_v, [rows, jnp.full((L,), 64, jnp.int32)],
                               emp_v[pl.ds(r0, L)])
            plsc.store_scatter(out_v, [rows, jnp.full((L,), 65, jnp.int32)],
                               lat_v[pl.ds(r0, L)])
            plsc.store_scatter(out_v, [rows, jnp.full((L,), 66, jnp.int32)],
                               lon_v[pl.ds(r0, L)])
            for k in range(SK):
                sk = plsc.load_gather(sk_v, [rows, jnp.full((L,), k, jnp.int32)])
                plsc.store_scatter(out_v, [rows, 67 + sk], ones)

        pltpu.sync_copy(out_v, out_hbm.at[pl.ds(base, R)])

        for g in range(R // L):
            r0 = g * L
            rows = r0 + iota
            for k in range(SK):
                sk = plsc.load_gather(sk_v, [rows, jnp.full((L,), k, jnp.int32)])
                plsc.store_scatter(out_v, [rows, 67 + sk], zeros)
        return 0

    lax.fori_loop(0, NCHUNK, do_chunk, 0)


@functools.partial(jax.jit, static_argnums=())
def _run(lid, ind, emp, lat, lon, sk, ltab, itab):
    mesh = plsc.VectorSubcoreMesh(core_axis_name="c", subcore_axis_name="s")
    f = functools.partial(
        pl.kernel,
        mesh=mesh,
        compiler_params=pltpu.CompilerParams(use_tc_tiling_on_sc=False,
                                             needs_layout_passes=False),
        out_type=jax.ShapeDtypeStruct((B, OUT_W), jnp.float32),
        scratch_types=[
            pltpu.VMEM((R,), jnp.int32),
            pltpu.VMEM((R,), jnp.int32),
            pltpu.VMEM((R,), jnp.float32),
            pltpu.VMEM((R,), jnp.float32),
            pltpu.VMEM((R,), jnp.float32),
            pltpu.VMEM((R, SK), jnp.int32),
            pltpu.VMEM((R, EMB), jnp.float32),
            pltpu.VMEM((R, EMB), jnp.float32),
            pltpu.VMEM((R, OUT_W), jnp.float32),
            pltpu.SemaphoreType.DMA,
        ],
    )(_sc_body)
    return f(lid, ind, emp, lat, lon, sk, ltab, itab)


def kernel(listing_id, listing_industry_type, employer_num_employees,
           listing_loc_latitude, listing_loc_longitude, listing_skills,
           listing_table, industry_table):
    return _run(listing_id.astype(jnp.int32),
                listing_industry_type.astype(jnp.int32),
                employer_num_employees,
                listing_loc_latitude,
                listing_loc_longitude,
                listing_skills.astype(jnp.int32),
                listing_table, industry_table)


# speedup vs baseline: 2.5513x; 1.1342x over previous
import functools

import jax
import jax.numpy as jnp
from jax import lax
from jax.experimental import pallas as pl
from jax.experimental.pallas import tpu as pltpu
from jax.experimental.pallas import tpu_sc as plsc

B = 16384
EMB = 32
SK = 20
OUT_W = 568
NI = OUT_W // 8
NJ = B // 128
NC = 2
NS = 16
L = 16
NW = NC * NS
JPW = NJ // NW
CH = 64


def _assemble(buf, lrows_v, irows_v, emp_v, lat_v, lon_v, sk_v, iota, ones):
    for g in range(CH // L):
        rows = g * L + iota
        for c in range(EMB):
            v = plsc.load_gather(lrows_v, [rows, jnp.full((L,), c, jnp.int32)])
            buf[c // 8, c % 8, pl.ds(g * L, L)] = v
        for c in range(EMB):
            v = plsc.load_gather(irows_v, [rows, jnp.full((L,), c, jnp.int32)])
            buf[4 + c // 8, c % 8, pl.ds(g * L, L)] = v
        buf[8, 0, pl.ds(g * L, L)] = emp_v[pl.ds(g * L, L)]
        buf[8, 1, pl.ds(g * L, L)] = lat_v[pl.ds(g * L, L)]
        buf[8, 2, pl.ds(g * L, L)] = lon_v[pl.ds(g * L, L)]
        for k in range(SK):
            sk = plsc.load_gather(sk_v, [rows, jnp.full((L,), k, jnp.int32)])
            c = 67 + sk
            plsc.store_scatter(buf, [lax.shift_right_logical(c, 3),
                                     lax.bitwise_and(c, 7), rows], ones)


def _clear(buf, sk_v, iota, zeros):
    for g in range(CH // L):
        rows = g * L + iota
        for k in range(SK):
            sk = plsc.load_gather(sk_v, [rows, jnp.full((L,), k, jnp.int32)])
            c = 67 + sk
            plsc.store_scatter(buf, [lax.shift_right_logical(c, 3),
                                     lax.bitwise_and(c, 7), rows], zeros)


def _sc_body(lid_hbm, ind_hbm, emp_hbm, lat_hbm, lon_hbm, sk_hbm,
             ltab_hbm, itab_hbm, out_hbm,
             lid_v, ind_v, emp_v, lat_v, lon_v, sk0_v, sk1_v,
             lrows_v, irows_v, buf0, buf1,
             sem_in, sem_g, sem_o0, sem_o1):
    wid = lax.axis_index("s") * NC + lax.axis_index("c")
    iota = lax.iota(jnp.int32, L)
    ones = jnp.full((L,), 1.0, jnp.float32)
    zeros = jnp.zeros((L,), jnp.float32)

    def zero_i(i, _):
        for k in range(8):
            for g in range(CH // L):
                buf0[i, k, pl.ds(g * L, L)] = zeros
                buf1[i, k, pl.ds(g * L, L)] = zeros
        return 0

    lax.fori_loop(8, NI, zero_i, 0)

    def do_chunk(jj, l0, buf, sk_v, sem_o, first):
        base = jj * 128 + l0
        @pl.when(jnp.logical_not(first))
        def _():
            pltpu.make_async_copy(
                buf, out_hbm.at[:, jj, :, pl.ds(l0, CH)], sem_o).wait()
            _clear(buf, sk_v, iota, zeros)

        cps = [
            pltpu.async_copy(lid_hbm.at[pl.ds(base, CH)], lid_v, sem_in),
            pltpu.async_copy(ind_hbm.at[pl.ds(base, CH)], ind_v, sem_in),
            pltpu.async_copy(emp_hbm.at[pl.ds(base, CH)], emp_v, sem_in),
            pltpu.async_copy(lat_hbm.at[pl.ds(base, CH)], lat_v, sem_in),
            pltpu.async_copy(lon_hbm.at[pl.ds(base, CH)], lon_v, sem_in),
            pltpu.async_copy(sk_hbm.at[pl.ds(base, CH)], sk_v, sem_in),
        ]
        for cp in cps:
            cp.wait()
        g1 = pltpu.async_copy(ltab_hbm.at[lid_v], lrows_v, sem_g)
        g2 = pltpu.async_copy(itab_hbm.at[ind_v], irows_v, sem_g)
        g1.wait()
        g2.wait()
        _assemble(buf, lrows_v, irows_v, emp_v, lat_v, lon_v, sk_v, iota, ones)
        pltpu.async_copy(buf, out_hbm.at[:, jj, :, pl.ds(l0, CH)], sem_o)

    def do_pair(i, _):
        jj = wid * JPW + i
        first = i == 0
        do_chunk(jj, 0, buf0, sk0_v, sem_o0, first)
        do_chunk(jj, CH, buf1, sk1_v, sem_o1, first)
        return 0

    lax.fori_loop(0, JPW, do_pair, 0)
    last_j = wid * JPW + JPW - 1
    pltpu.make_async_copy(
        buf0, out_hbm.at[:, last_j, :, pl.ds(0, CH)], sem_o0).wait()
    pltpu.make_async_copy(
        buf1, out_hbm.at[:, last_j, :, pl.ds(CH, CH)], sem_o1).wait()


@jax.jit
def _run(lid, ind, emp, lat, lon, sk, ltab, itab):
    mesh = plsc.VectorSubcoreMesh(core_axis_name="c", subcore_axis_name="s")
    f = functools.partial(
        pl.kernel,
        mesh=mesh,
        compiler_params=pltpu.CompilerParams(use_tc_tiling_on_sc=False,
                                             needs_layout_passes=False),
        out_type=jax.ShapeDtypeStruct((NI, NJ, 8, 128), jnp.float32),
        scratch_types=[
            pltpu.VMEM((CH,), jnp.int32),
            pltpu.VMEM((CH,), jnp.int32),
            pltpu.VMEM((CH,), jnp.float32),
            pltpu.VMEM((CH,), jnp.float32),
            pltpu.VMEM((CH,), jnp.float32),
            pltpu.VMEM((CH, SK), jnp.int32),
            pltpu.VMEM((CH, SK), jnp.int32),
            pltpu.VMEM((CH, EMB), jnp.float32),
            pltpu.VMEM((CH, EMB), jnp.float32),
            pltpu.VMEM((NI, 8, CH), jnp.float32),
            pltpu.VMEM((NI, 8, CH), jnp.float32),
            pltpu.SemaphoreType.DMA,
            pltpu.SemaphoreType.DMA,
            pltpu.SemaphoreType.DMA,
            pltpu.SemaphoreType.DMA,
        ],
    )(_sc_body)
    out4 = f(lid, ind, emp, lat, lon, sk, ltab, itab)
    return jnp.transpose(out4, (1, 3, 0, 2)).reshape(B, OUT_W)


def kernel(listing_id, listing_industry_type, employer_num_employees,
           listing_loc_latitude, listing_loc_longitude, listing_skills,
           listing_table, industry_table):
    return _run(listing_id.astype(jnp.int32),
                listing_industry_type.astype(jnp.int32),
                employer_num_employees,
                listing_loc_latitude,
                listing_loc_longitude,
                listing_skills.astype(jnp.int32),
                listing_table, industry_table)
